# baseline (speedup 1.0000x reference)
import jax
import jax.numpy as jnp
from jax.experimental import pallas as pl
from jax.experimental.pallas import tpu as pltpu

NUM_EXPERTS = 64
TOP_K = 8
BLOCK_T = 256
N_STREAMS = 4
CHUNK_T = 64


def _router_part(x, wt, j, logits_ref, scores_ref, ew_ref, ei_ref):
    logits = jnp.dot(x, wt, preferred_element_type=jnp.float32)
    m = jnp.max(logits, axis=-1, keepdims=True)
    e = jnp.exp(logits - m)
    scores = e / jnp.sum(e, axis=-1, keepdims=True)
    logits_ref[j, 0] = logits
    scores_ref[j, 0] = scores

    for r in range(0, BLOCK_T, CHUNK_T):
        s = scores[r:r + CHUNK_T]
        colf = jax.lax.broadcasted_iota(
            jnp.int32, s.shape, 1).astype(jnp.float32)
        big = jnp.float32(NUM_EXPERTS)
        vals = []
        idxs = []
        for _ in range(TOP_K):
            mk = jnp.max(s, axis=-1, keepdims=True)
            ik = jnp.min(jnp.where(s == mk, colf, big), axis=-1, keepdims=True)
            vals.append(mk)
            idxs.append(ik)
            s = jnp.where(colf == ik, jnp.float32(-1.0), s)
        tv = jnp.concatenate(vals, axis=-1)
        ti = jnp.concatenate(idxs, axis=-1)
        ee = jnp.exp(tv - tv[:, :1])
        ew_ref[j, 0, r:r + CHUNK_T] = ee / jnp.sum(ee, axis=-1, keepdims=True)
        ei_ref[j, 0, r:r + CHUNK_T] = ti.astype(jnp.int32)


def _router_block(*refs):
    x_refs = refs[:N_STREAMS]
    wt_ref = refs[N_STREAMS]
    logits_ref, scores_ref, ew_ref, ei_ref = refs[N_STREAMS + 1:]
    wt = wt_ref[...]
    for j, x_ref in enumerate(x_refs):
        _router_part(x_ref[...], wt, j,
                     logits_ref, scores_ref, ew_ref, ei_ref)


def kernel(x, W):
    bs, sq, d = x.shape
    n_tok = bs * sq
    x2 = x.reshape(n_tok, d)
    wt = W.T
    ns = N_STREAMS
    n_steps = n_tok // (ns * BLOCK_T)
    E, K = NUM_EXPERTS, TOP_K

    def xmap(j):
        return lambda i: (j * n_steps + i, 0)

    def omap(i):
        return (0, i, 0, 0)

    logits, scores, ew, ei = pl.pallas_call(
        _router_block,
        grid=(n_steps,),
        in_specs=[pl.BlockSpec((BLOCK_T, d), xmap(j)) for j in range(ns)]
        + [pl.BlockSpec((d, E), lambda i: (0, 0))],
        out_specs=(
            pl.BlockSpec((ns, 1, BLOCK_T, E), omap),
            pl.BlockSpec((ns, 1, BLOCK_T, E), omap),
            pl.BlockSpec((ns, 1, BLOCK_T, K), omap),
            pl.BlockSpec((ns, 1, BLOCK_T, K), omap),
        ),
        out_shape=(
            jax.ShapeDtypeStruct((ns, n_steps, BLOCK_T, E), jnp.float32),
            jax.ShapeDtypeStruct((ns, n_steps, BLOCK_T, E), jnp.float32),
            jax.ShapeDtypeStruct((ns, n_steps, BLOCK_T, K), jnp.float32),
            jax.ShapeDtypeStruct((ns, n_steps, BLOCK_T, K), jnp.int32),
        ),
        compiler_params=pltpu.CompilerParams(
            dimension_semantics=("parallel",)),
    )(*([x2] * ns), wt)
    return (scores.reshape(n_tok, E), logits.reshape(n_tok, E),
            ew.reshape(n_tok, K), ei.reshape(n_tok, K))


# speedup vs baseline: 1.0623x; 1.0623x over previous
import jax
import jax.numpy as jnp
from jax.experimental import pallas as pl
from jax.experimental.pallas import tpu as pltpu

NUM_EXPERTS = 64
TOP_K = 8
BLOCK_T = 256
N_STREAMS = 4
CHUNK_T = 64


def _router_part(x, wt, j, logits_ref, scores_ref, ew_ref, ei_ref):
    logits = jnp.dot(x, wt, preferred_element_type=jnp.float32)
    m = jnp.max(logits, axis=-1, keepdims=True)
    e = jnp.exp(logits - m)
    scores = e / jnp.sum(e, axis=-1, keepdims=True)
    logits_ref[j, 0] = logits
    scores_ref[j, 0] = scores

    st = scores.T
    rowf = jax.lax.broadcasted_iota(
        jnp.int32, st.shape, 0).astype(jnp.float32)
    big = jnp.float32(NUM_EXPERTS)
    vals = []
    idxs = []
    for _ in range(TOP_K):
        mk = jnp.max(st, axis=0, keepdims=True)
        ik = jnp.min(jnp.where(st == mk, rowf, big), axis=0, keepdims=True)
        vals.append(mk)
        idxs.append(ik)
        st = jnp.where(rowf == ik, jnp.float32(-1.0), st)
    tv = jnp.concatenate(vals, axis=0)
    ti = jnp.concatenate(idxs, axis=0)
    ee = jnp.exp(tv - tv[:1])
    ew = ee / jnp.sum(ee, axis=0, keepdims=True)
    ew_ref[j, 0] = ew.T
    ei_ref[j, 0] = ti.T.astype(jnp.int32)


def _router_block(*refs):
    x_refs = refs[:N_STREAMS]
    wt_ref = refs[N_STREAMS]
    logits_ref, scores_ref, ew_ref, ei_ref = refs[N_STREAMS + 1:]
    wt = wt_ref[...]
    for j, x_ref in enumerate(x_refs):
        _router_part(x_ref[...], wt, j,
                     logits_ref, scores_ref, ew_ref, ei_ref)


def kernel(x, W):
    bs, sq, d = x.shape
    n_tok = bs * sq
    x2 = x.reshape(n_tok, d)
    wt = W.T
    ns = N_STREAMS
    n_steps = n_tok // (ns * BLOCK_T)
    E, K = NUM_EXPERTS, TOP_K

    def xmap(j):
        return lambda i: (j * n_steps + i, 0)

    def omap(i):
        return (0, i, 0, 0)

    logits, scores, ew, ei = pl.pallas_call(
        _router_block,
        grid=(n_steps,),
        in_specs=[pl.BlockSpec((BLOCK_T, d), xmap(j)) for j in range(ns)]
        + [pl.BlockSpec((d, E), lambda i: (0, 0))],
        out_specs=(
            pl.BlockSpec((ns, 1, BLOCK_T, E), omap),
            pl.BlockSpec((ns, 1, BLOCK_T, E), omap),
            pl.BlockSpec((ns, 1, BLOCK_T, K), omap),
            pl.BlockSpec((ns, 1, BLOCK_T, K), omap),
        ),
        out_shape=(
            jax.ShapeDtypeStruct((ns, n_steps, BLOCK_T, E), jnp.float32),
            jax.ShapeDtypeStruct((ns, n_steps, BLOCK_T, E), jnp.float32),
            jax.ShapeDtypeStruct((ns, n_steps, BLOCK_T, K), jnp.float32),
            jax.ShapeDtypeStruct((ns, n_steps, BLOCK_T, K), jnp.int32),
        ),
        compiler_params=pltpu.CompilerParams(
            dimension_semantics=("parallel",)),
    )(*([x2] * ns), wt)
    return (scores.reshape(n_tok, E), logits.reshape(n_tok, E),
            ew.reshape(n_tok, K), ei.reshape(n_tok, K))


# speedup vs baseline: 1.0769x; 1.0138x over previous
import jax
import jax.numpy as jnp
from jax.experimental import pallas as pl
from jax.experimental.pallas import tpu as pltpu

NUM_EXPERTS = 64
TOP_K = 8
BLOCK_T = 256
N_STREAMS = 4


def _router_part(x, wt, j, logits_ref, scores_ref, ew_ref, ei_ref):
    logits = jnp.dot(x, wt, preferred_element_type=jnp.float32)
    m = jnp.max(logits, axis=-1, keepdims=True)
    e = jnp.exp(logits - m)
    scores = e / jnp.sum(e, axis=-1, keepdims=True)
    logits_ref[j, 0] = logits
    scores_ref[j, 0] = scores

    st = scores.T
    rowf = jax.lax.broadcasted_iota(
        jnp.int32, st.shape, 0).astype(jnp.float32)
    big = jnp.float32(NUM_EXPERTS)
    vals = []
    idxs = []
    for _ in range(TOP_K):
        mk = jnp.max(st, axis=0, keepdims=True)
        ik = jnp.min(jnp.where(st == mk, rowf, big), axis=0, keepdims=True)
        vals.append(mk)
        idxs.append(ik)
        st = jnp.where(rowf == ik, jnp.float32(-1.0), st)
    tv = jnp.concatenate(vals, axis=0)
    ti = jnp.concatenate(idxs, axis=0)
    ee = jnp.exp(tv - tv[:1])
    ew = ee / jnp.sum(ee, axis=0, keepdims=True)
    ew_ref[j, 0] = ew.T
    ei_ref[j, 0] = ti.T.astype(jnp.int32)


def _router_block(*refs):
    x_refs = refs[:N_STREAMS]
    wt_ref = refs[N_STREAMS]
    logits_ref, scores_ref, ew_ref, ei_ref = refs[N_STREAMS + 1:]
    wt = wt_ref[...]
    for j, x_ref in enumerate(x_refs):
        _router_part(x_ref[...], wt, j,
                     logits_ref, scores_ref, ew_ref, ei_ref)


def kernel(x, W):
    bs, sq, d = x.shape
    n_tok = bs * sq
    x2 = x.reshape(n_tok, d)
    wt = W.T
    ns = N_STREAMS
    n_steps = n_tok // (ns * BLOCK_T)
    E, K = NUM_EXPERTS, TOP_K

    def xmap(j):
        return lambda i: (j * n_steps + i, 0)

    def omap(i):
        return (0, i, 0, 0)

    logits, scores, ew, ei = pl.pallas_call(
        _router_block,
        grid=(n_steps,),
        in_specs=[pl.BlockSpec((BLOCK_T, d), xmap(j)) for j in range(ns)]
        + [pl.BlockSpec((d, E), lambda i: (0, 0))],
        out_specs=(
            pl.BlockSpec((ns, 1, BLOCK_T, E), omap),
            pl.BlockSpec((ns, 1, BLOCK_T, E), omap),
            pl.BlockSpec((ns, 1, BLOCK_T, K), omap),
            pl.BlockSpec((ns, 1, BLOCK_T, K), omap),
        ),
        out_shape=(
            jax.ShapeDtypeStruct((ns, n_steps, BLOCK_T, E), jnp.float32),
            jax.ShapeDtypeStruct((ns, n_steps, BLOCK_T, E), jnp.float32),
            jax.ShapeDtypeStruct((ns, n_steps, BLOCK_T, K), jnp.float32),
            jax.ShapeDtypeStruct((ns, n_steps, BLOCK_T, K), jnp.int32),
        ),
        compiler_params=pltpu.CompilerParams(
            dimension_semantics=("parallel",)),
    )(*([x2] * ns), wt)
    return (scores.reshape(n_tok, E), logits.reshape(n_tok, E),
            ew.reshape(n_tok, K), ei.reshape(n_tok, K))
